# baseline (speedup 1.0000x reference)
import functools

import jax
import jax.numpy as jnp
from jax import lax
from jax.experimental import pallas as pl
from jax.experimental.pallas import tpu as pltpu
from jax.experimental.pallas import tpu_sc as plsc

_LANES = 16
_NUM_SUBCORES = 16
_B = 4096
_BPW = _B // _NUM_SUBCORES
_HALF = _BPW // 2
_TABLE = 1001

_mesh = plsc.VectorSubcoreMesh(
    core_axis_name="c", subcore_axis_name="s", num_cores=1, num_subcores=16
)


@functools.partial(
    pl.kernel,
    out_type=jax.ShapeDtypeStruct((_B,), jnp.float32),
    mesh=_mesh,
    scratch_types=[
        pltpu.VMEM((_TABLE,), jnp.float32),
        pltpu.VMEM((_BPW,), jnp.int32),
        pltpu.VMEM((_BPW,), jnp.float32),
        pltpu.SemaphoreType.DMA,
        pltpu.SemaphoreType.DMA,
        pltpu.SemaphoreType.DMA,
    ],
    compiler_params=pltpu.CompilerParams(needs_layout_passes=False),
)
def _gather_sc(
    betas_hbm, t_hbm, out_hbm, table_v, idx_v, out_v, sem_t, sem_i, sem_o
):
    wid = lax.axis_index("s")
    base = wid * _BPW
    tbl_cp = pltpu.async_copy(betas_hbm, table_v, sem_t)
    idx_cp = pltpu.async_copy(t_hbm.at[pl.ds(base, _BPW)], idx_v, sem_i)
    idx_cp.wait()
    tbl_cp.wait()
    for j in range(_HALF // _LANES):
        idx = idx_v[pl.ds(j * _LANES, _LANES)]
        out_v[pl.ds(j * _LANES, _LANES)] = plsc.load_gather(table_v, [idx])
    out0 = pltpu.async_copy(
        out_v.at[pl.ds(0, _HALF)], out_hbm.at[pl.ds(base, _HALF)], sem_o
    )
    for j in range(_HALF // _LANES, _BPW // _LANES):
        idx = idx_v[pl.ds(j * _LANES, _LANES)]
        out_v[pl.ds(j * _LANES, _LANES)] = plsc.load_gather(table_v, [idx])
    out1 = pltpu.async_copy(
        out_v.at[pl.ds(_HALF, _HALF)], out_hbm.at[pl.ds(base + _HALF, _HALF)], sem_o
    )
    out0.wait()
    out1.wait()


def kernel(betas, t_int):
    return _gather_sc(betas.astype(jnp.float32), t_int.astype(jnp.int32))


# speedup vs baseline: 1.0009x; 1.0009x over previous
import functools

import jax
import jax.numpy as jnp
from jax import lax
from jax.experimental import pallas as pl
from jax.experimental.pallas import tpu as pltpu
from jax.experimental.pallas import tpu_sc as plsc

_LANES = 16
_NUM_SUBCORES = 16
_B = 4096
_BPW = _B // _NUM_SUBCORES
_TABLE = 1001

_mesh = plsc.VectorSubcoreMesh(
    core_axis_name="c", subcore_axis_name="s", num_cores=1, num_subcores=16
)


@functools.partial(
    pl.kernel,
    out_type=jax.ShapeDtypeStruct((_B,), jnp.float32),
    mesh=_mesh,
    scratch_types=[
        pltpu.VMEM((_TABLE,), jnp.float32),
        pltpu.VMEM((_BPW,), jnp.int32),
        pltpu.VMEM((_BPW,), jnp.float32),
        pltpu.SemaphoreType.DMA,
    ],
    compiler_params=pltpu.CompilerParams(needs_layout_passes=False),
)
def _gather_sc(betas_hbm, t_hbm, out_hbm, table_v, idx_v, out_v, sem):
    wid = lax.axis_index("s")
    base = wid * _BPW
    tbl_cp = pltpu.async_copy(betas_hbm, table_v, sem)
    idx_cp = pltpu.async_copy(t_hbm.at[pl.ds(base, _BPW)], idx_v, sem)
    idx_cp.wait()
    tbl_cp.wait()
    for j in range(_BPW // _LANES):
        idx = idx_v[pl.ds(j * _LANES, _LANES)]
        out_v[pl.ds(j * _LANES, _LANES)] = plsc.load_gather(table_v, [idx])
    pltpu.sync_copy(out_v, out_hbm.at[pl.ds(base, _BPW)])


def kernel(betas, t_int):
    return _gather_sc(betas.astype(jnp.float32), t_int.astype(jnp.int32))


# speedup vs baseline: 1.0062x; 1.0053x over previous
import functools

import jax
import jax.numpy as jnp
from jax import lax
from jax.experimental import pallas as pl
from jax.experimental.pallas import tpu as pltpu
from jax.experimental.pallas import tpu_sc as plsc

_LANES = 16
_NUM_SUBCORES = 8
_B = 4096
_BPW = _B // _NUM_SUBCORES
_TABLE = 1001

_mesh = plsc.VectorSubcoreMesh(
    core_axis_name="c", subcore_axis_name="s", num_cores=1, num_subcores=8
)


@functools.partial(
    pl.kernel,
    out_type=jax.ShapeDtypeStruct((_B,), jnp.float32),
    mesh=_mesh,
    scratch_types=[
        pltpu.VMEM((_TABLE,), jnp.float32),
        pltpu.VMEM((_BPW,), jnp.int32),
        pltpu.VMEM((_BPW,), jnp.float32),
        pltpu.SemaphoreType.DMA,
    ],
    compiler_params=pltpu.CompilerParams(needs_layout_passes=False),
)
def _gather_sc(betas_hbm, t_hbm, out_hbm, table_v, idx_v, out_v, sem):
    wid = lax.axis_index("s")
    base = wid * _BPW
    tbl_cp = pltpu.async_copy(betas_hbm, table_v, sem)
    idx_cp = pltpu.async_copy(t_hbm.at[pl.ds(base, _BPW)], idx_v, sem)
    idx_cp.wait()
    tbl_cp.wait()
    for j in range(_BPW // _LANES):
        idx = idx_v[pl.ds(j * _LANES, _LANES)]
        out_v[pl.ds(j * _LANES, _LANES)] = plsc.load_gather(table_v, [idx])
    pltpu.sync_copy(out_v, out_hbm.at[pl.ds(base, _BPW)])


def kernel(betas, t_int):
    return _gather_sc(betas.astype(jnp.float32), t_int.astype(jnp.int32))
